# Initial kernel scaffold; baseline (speedup 1.0000x reference)
import jax
import jax.numpy as jnp
from jax.experimental import pallas as pl


def kernel(in_feat, edge_index, W1, b1, W2, b2, Wc, bc):
    raise NotImplementedError("write your pallas kernel here")


# speedup vs baseline: 9.7756x; 9.7756x over previous
import functools

import jax
import jax.numpy as jnp
from jax import lax
from jax.experimental import pallas as pl
from jax.experimental.pallas import tpu as pltpu
from jax.experimental.pallas import tpu_sc as plsc

N = 10000
E = 320000
H = 128
NP = 10240
NC = 2
NS = 16
K = 80
ROWS = E // K
ROWS_PER_SC = ROWS // NC
CH = ROWS_PER_SC // NS
RPT = NP // NS

_mesh = plsc.VectorSubcoreMesh(core_axis_name="c", subcore_axis_name="s")
_f32 = jnp.float32


def _deg_body(srcr, dstr, z1, degout, degin,
              srcb, dstb, ones_v, go_sp, gi_sp):
    c = lax.axis_index("c")
    s = lax.axis_index("s")
    wid = c * NS + s
    pltpu.sync_copy(z1.at[pl.ds(s * RPT, RPT)], go_sp.at[pl.ds(s * RPT, RPT)])
    pltpu.sync_copy(z1.at[pl.ds(s * RPT, RPT)], gi_sp.at[pl.ds(s * RPT, RPT)])
    pltpu.sync_copy(srcr.at[wid], srcb)
    pltpu.sync_copy(dstr.at[wid], dstb)
    for k in range(K // 16):
        ones_v[pl.ds(k * 16, 16)] = jnp.ones((16,), _f32)
    plsc.subcore_barrier()

    def chunk(j, carry):
        pltpu.sync_copy(ones_v, go_sp.at[srcb.at[j]], add=True)
        pltpu.sync_copy(ones_v, gi_sp.at[dstb.at[j]], add=True)
        return carry

    lax.fori_loop(0, CH, chunk, 0, unroll=False)
    plsc.subcore_barrier()

    @pl.when(s == 0)
    def _():
        pltpu.sync_copy(go_sp, degout.at[c])
        pltpu.sync_copy(gi_sp, degin.at[c])


def _deg_call(srcr, dstr, z1):
    return pl.kernel(
        _deg_body,
        out_type=(
            jax.ShapeDtypeStruct((NC, NP), _f32),
            jax.ShapeDtypeStruct((NC, NP), _f32),
        ),
        mesh=_mesh,
        scratch_types=dict(
            srcb=pltpu.VMEM((CH, K), jnp.int32),
            dstb=pltpu.VMEM((CH, K), jnp.int32),
            ones_v=pltpu.VMEM((K,), _f32),
            go_sp=pltpu.VMEM_SHARED((NP,), _f32),
            gi_sp=pltpu.VMEM_SHARED((NP,), _f32),
        ),
    )(srcr, dstr, z1)


def _norm_mm_body(x_ref, w1_ref, dgo_ref, dgi_ref,
                  hsc_ref, nsrc_ref, ndst_ref):
    dgo = dgo_ref[:, 0:1] + dgo_ref[:, 1:2]
    dgi = dgi_ref[:, 0:1] + dgi_ref[:, 1:2]
    nsrc = lax.rsqrt(jnp.maximum(dgo, 1.0))
    ndst = lax.rsqrt(jnp.maximum(dgi, 1.0))
    nsrc_ref[...] = nsrc
    ndst_ref[...] = ndst
    xw = jnp.dot(x_ref[...], w1_ref[...], preferred_element_type=_f32)
    hsc_ref[...] = xw * nsrc[:N]


def _norm_mm_call(x, w1, dgo_t, dgi_t):
    return pl.pallas_call(
        _norm_mm_body,
        out_shape=(
            jax.ShapeDtypeStruct((N, H), _f32),
            jax.ShapeDtypeStruct((NP, 1), _f32),
            jax.ShapeDtypeStruct((NP, 1), _f32),
        ),
    )(x, w1, dgo_t, dgi_t)


def _agg_body(srcr, dstr, hsc, ndst1, z1, z2, agg, wout,
              srcb, dstb, stage, wvals, acc_sp, w_sp, sem):
    c = lax.axis_index("c")
    s = lax.axis_index("s")
    wid = c * NS + s
    pltpu.sync_copy(z2.at[pl.ds(s * RPT, RPT)], acc_sp.at[pl.ds(s * RPT, RPT)])
    pltpu.sync_copy(z1.at[pl.ds(s * RPT, RPT)], w_sp.at[pl.ds(s * RPT, RPT)])
    pltpu.sync_copy(srcr.at[wid], srcb)
    pltpu.sync_copy(dstr.at[wid], dstb)
    plsc.subcore_barrier()

    def chunk(j, carry):
        drow = dstb.at[j]
        srow = srcb.at[j]
        pltpu.async_copy(hsc.at[srow], stage, sem).wait()
        pltpu.async_copy(ndst1.at[drow], wvals, sem).wait()
        pltpu.sync_copy(stage, acc_sp.at[drow], add=True)
        pltpu.sync_copy(wvals, w_sp.at[srow], add=True)
        return carry

    lax.fori_loop(0, CH, chunk, 0, unroll=False)
    plsc.subcore_barrier()

    pltpu.sync_copy(acc_sp.at[pl.ds(s * RPT, RPT)],
                    agg.at[c, pl.ds(s * RPT, RPT)])

    @pl.when(s == 0)
    def _():
        pltpu.sync_copy(w_sp, wout.at[c])


def _agg_call(srcr, dstr, hsc, ndst1, z1, z2):
    return pl.kernel(
        _agg_body,
        out_type=(
            jax.ShapeDtypeStruct((NC, NP, H), _f32),
            jax.ShapeDtypeStruct((NC, NP), _f32),
        ),
        mesh=_mesh,
        scratch_types=dict(
            srcb=pltpu.VMEM((CH, K), jnp.int32),
            dstb=pltpu.VMEM((CH, K), jnp.int32),
            stage=pltpu.VMEM((K, H), _f32),
            wvals=pltpu.VMEM((K,), _f32),
            acc_sp=pltpu.VMEM_SHARED((NP, H), _f32),
            w_sp=pltpu.VMEM_SHARED((NP,), _f32),
            sem=pltpu.SemaphoreType.DMA,
        ),
    )(srcr, dstr, hsc, ndst1, z1, z2)


def _final_body(agg_ref, ndst_ref, nsrc_ref, wp_ref,
                b1_ref, w2_ref, b2_ref, wc_ref, bc_ref, out_ref):
    agg = agg_ref[0] + agg_ref[1]
    h1 = jnp.maximum(agg * ndst_ref[...] + b1_ref[...], 0.0)
    wsum = wp_ref[:, 0:1] + wp_ref[:, 1:2]
    coeff = wsum * nsrc_ref[...]
    u = lax.dot_general(coeff, h1, (((0,), (0,)), ((), ())),
                        preferred_element_type=_f32)
    hg = jnp.dot(u, w2_ref[...], preferred_element_type=_f32) * (1.0 / N)
    hg = hg + b2_ref[...]
    out_ref[...] = jnp.dot(hg, wc_ref[...], preferred_element_type=_f32) \
        + bc_ref[...]


def _final_call(agg, ndst, nsrc, wp_t, b1, w2, b2, wc, bc):
    return pl.pallas_call(
        _final_body,
        out_shape=jax.ShapeDtypeStruct((1, 10), _f32),
    )(agg, ndst, nsrc, wp_t, b1, w2, b2, wc, bc)


@jax.jit
def kernel(in_feat, edge_index, W1, b1, W2, b2, Wc, bc):
    srcr = edge_index[0].reshape(NC * NS, CH, K)
    dstr = edge_index[1].reshape(NC * NS, CH, K)
    z1 = jnp.zeros((NP,), _f32)
    z2 = jnp.zeros((NP, H), _f32)

    degout, degin = _deg_call(srcr, dstr, z1)
    hsc, nsrc, ndst = _norm_mm_call(in_feat, W1, degout.T, degin.T)
    agg, w_parts = _agg_call(srcr, dstr, hsc, ndst[:, 0], z1, z2)
    return _final_call(agg, ndst, nsrc, w_parts.T,
                       b1.reshape(1, H), W2, b2.reshape(1, H),
                       Wc.reshape(H, 10), bc.reshape(1, 10))
